# baseline (speedup 1.0000x reference)
import functools

import jax
import jax.numpy as jnp
from jax import lax
from jax.experimental import pallas as pl
from jax.experimental.pallas import tpu as pltpu
from jax.experimental.pallas import tpu_sc as plsc

N = 10000
E = 320000
D = 128
DH = 32
S = 8192
LEAD = 48

NC = 2
NS = 16
NW = NC * NS

CB = 128
CPW = 80
EP = NW * CPW * CB
PSH = 14
NPAD = 10112
NPS = NPAD // NS
SEG_CPT = EP // CB // NS
SEG_HALF = SEG_CPT // 2


def _wid():
    return lax.axis_index("c") * NS + lax.axis_index("s")


def _sc_deg_body(row2_hbm, zn_hbm, out_hbm, idx_v, ones_v, acc):
    c = lax.axis_index("c")
    s = lax.axis_index("s")
    w = _wid()

    for i in range(CB // 16):
        ones_v[pl.ds(i * 16, 16)] = jnp.ones((16,), jnp.float32)

    @pl.when(s == 0)
    def _():
        pltpu.sync_copy(zn_hbm, acc)

    pltpu.sync_copy(row2_hbm.at[pl.ds(w * CPW, CPW), :], idx_v)
    plsc.subcore_barrier()

    @pl.loop(0, CPW)
    def _(j):
        pltpu.sync_copy(ones_v, acc.at[idx_v.at[j]], add=True)

    plsc.subcore_barrier()

    @pl.when(s == 0)
    def _():
        pltpu.sync_copy(acc, out_hbm.at[pl.ds(c * NPAD, NPAD)])


def _sc_seg_body(tbl_hbm, pidx_hbm, z2_hbm, out_hbm, pidx, rv0, rv1, cv,
                 buf0, buf1, acc, sg0, sg1):
    c = lax.axis_index("c")
    s = lax.axis_index("s")

    def unpack_rows(j, rv):
        for k in range(CB // 16):
            v = pidx[j, pl.ds(16 * k, 16)]
            rv[pl.ds(16 * k, 16)] = lax.shift_right_logical(v, PSH)

    def unpack_cols(j):
        for k in range(CB // 16):
            v = pidx[j, pl.ds(16 * k, 16)]
            cv[pl.ds(16 * k, 16)] = lax.bitwise_and(v, (1 << PSH) - 1)

    def run(base, cpw):
        pltpu.sync_copy(pidx_hbm.at[pl.ds(base, cpw), :], pidx.at[pl.ds(0, cpw), :])
        unpack_rows(0, rv0)
        pltpu.async_copy(tbl_hbm.at[rv0], buf0, sg0)
        unpack_rows(1, rv1)
        pltpu.async_copy(tbl_hbm.at[rv1], buf1, sg1)

        @pl.loop(0, cpw // 2)
        def _(i):
            j = 2 * i
            pltpu.make_async_copy(tbl_hbm.at[rv0], buf0, sg0).wait()
            unpack_cols(j)
            pltpu.sync_copy(buf0, acc.at[cv], add=True)

            @pl.when(j + 2 < cpw)
            def _():
                unpack_rows(j + 2, rv0)
                pltpu.async_copy(tbl_hbm.at[rv0], buf0, sg0)

            pltpu.make_async_copy(tbl_hbm.at[rv1], buf1, sg1).wait()
            unpack_cols(j + 1)
            pltpu.sync_copy(buf1, acc.at[cv], add=True)

            @pl.when(j + 3 < cpw)
            def _():
                unpack_rows(j + 3, rv1)
                pltpu.async_copy(tbl_hbm.at[rv1], buf1, sg1)

    @pl.when(c == 0)
    def _():
        pltpu.sync_copy(
            z2_hbm.at[pl.ds(s * NPS, NPS), :], acc.at[pl.ds(s * NPS, NPS), :]
        )
        run(s * SEG_CPT, 128)
        run(s * SEG_CPT + 128, 32)

    plsc.subcore_barrier()

    @pl.when(c == 0)
    def _():
        pltpu.sync_copy(
            acc.at[pl.ds(s * NPS, NPS), :], out_hbm.at[pl.ds(s * NPS, NPS), :]
        )


_PPW = S // NW // 128


def _sc_pair_body(h1_hbm, sd_hbm, outL_hbm, outR_hbm, idx_v, buf, sem):
    w = _wid()
    pltpu.sync_copy(sd_hbm, idx_v)
    for t in range(2):
        out = outL_hbm if t == 0 else outR_hbm
        for j in range(_PPW):
            r = t * (S // 128) + w * _PPW + j
            pltpu.async_copy(h1_hbm.at[idx_v.at[r]], buf, sem).wait()
            pltpu.sync_copy(buf, out.at[pl.ds(w * _PPW * 128 + j * 128, 128), :])


@functools.cache
def _sc_kernels():
    mesh = plsc.VectorSubcoreMesh(
        core_axis_name="c", subcore_axis_name="s", num_cores=NC, num_subcores=NS
    )
    sc_deg = pl.kernel(
        _sc_deg_body,
        out_type=jax.ShapeDtypeStruct((NC * NPAD,), jnp.float32),
        mesh=mesh,
        scratch_types=[
            pltpu.VMEM((CPW, CB), jnp.int32),
            pltpu.VMEM((CB,), jnp.float32),
            pltpu.VMEM_SHARED((NPAD,), jnp.float32),
        ],
    )
    sc_seg = pl.kernel(
        _sc_seg_body,
        out_type=jax.ShapeDtypeStruct((NPAD, D), jnp.float32),
        mesh=mesh,
        scratch_types=[
            pltpu.VMEM((128, CB), jnp.int32),
            pltpu.VMEM((CB,), jnp.int32),
            pltpu.VMEM((CB,), jnp.int32),
            pltpu.VMEM((CB,), jnp.int32),
            pltpu.VMEM((CB, D), jnp.float32),
            pltpu.VMEM((CB, D), jnp.float32),
            pltpu.VMEM_SHARED((NPAD, D), jnp.float32),
            pltpu.SemaphoreType.DMA,
            pltpu.SemaphoreType.DMA,
        ],
    )
    sc_pair = pl.kernel(
        _sc_pair_body,
        out_type=(
            jax.ShapeDtypeStruct((S, D), jnp.float32),
            jax.ShapeDtypeStruct((S, D), jnp.float32),
        ),
        mesh=mesh,
        scratch_types=[
            pltpu.VMEM((2 * (S // 128), 128), jnp.int32),
            pltpu.VMEM((128, D), jnp.float32),
            pltpu.SemaphoreType.DMA,
        ],
    )
    return sc_deg, sc_seg, sc_pair


_RB = 400
_GRID_N = N // _RB


def _sigmoid(v):
    return 1.0 / (1.0 + jnp.exp(-v))


def _tc_scale_body(degp_ref, x_ref, xs_ref, disc_ref):
    deg = jnp.sum(degp_ref[...], axis=1, keepdims=True)
    good = deg > 0.0
    dis = jnp.where(good, jax.lax.rsqrt(jnp.where(good, deg, 1.0)), 0.0)
    disc_ref[...] = dis
    xs_ref[...] = dis * x_ref[...]


_tc_scale = pl.pallas_call(
    _tc_scale_body,
    grid=(_GRID_N,),
    in_specs=[
        pl.BlockSpec((_RB, 2), lambda i: (i, 0)),
        pl.BlockSpec((_RB, D), lambda i: (i, 0)),
    ],
    out_specs=[
        pl.BlockSpec((_RB, D), lambda i: (i, 0)),
        pl.BlockSpec((_RB, 1), lambda i: (i, 0)),
    ],
    out_shape=[
        jax.ShapeDtypeStruct((N, D), jnp.float32),
        jax.ShapeDtypeStruct((N, 1), jnp.float32),
    ],
)


def _tc_mid_body(s1p_ref, disc_ref, t1_ref, u1_ref):
    P = s1p_ref[...]
    dis = disc_ref[...]
    t1 = -dis * P
    t1_ref[...] = t1
    u1_ref[...] = dis * t1


_tc_mid = pl.pallas_call(
    _tc_mid_body,
    grid=(_GRID_N,),
    in_specs=[
        pl.BlockSpec((_RB, D), lambda i: (i, 0)),
        pl.BlockSpec((_RB, 1), lambda i: (i, 0)),
    ],
    out_specs=[
        pl.BlockSpec((_RB, D), lambda i: (i, 0)),
        pl.BlockSpec((_RB, D), lambda i: (i, 0)),
    ],
    out_shape=[
        jax.ShapeDtypeStruct((N, D), jnp.float32),
        jax.ShapeDtypeStruct((N, D), jnp.float32),
    ],
)


def _tc_gate_body(x_ref, t1_ref, s2p_ref, disc_ref, a0_ref, a1_ref, a2_ref,
                  consts_ref, f1_ref, h1_ref):
    x = x_ref[...]
    dis = disc_ref[...]
    t2 = -2.0 * dis * s2p_ref[...] - x
    pre = (
        jnp.dot(x, a0_ref[...], preferred_element_type=jnp.float32)
        + jnp.dot(t1_ref[...], a1_ref[...], preferred_element_type=jnp.float32)
        + jnp.dot(t2, a2_ref[...], preferred_element_type=jnp.float32)
        + consts_ref[0:1, :]
    )
    gi = _sigmoid(pre[:, 0:DH])
    gt = jnp.tanh(pre[:, DH:2 * DH])
    cc = gi * gt
    go = _sigmoid(pre[:, 2 * DH:3 * DH] + consts_ref[1:2, 2 * DH:3 * DH] * cc)
    h = jnp.maximum(go * jnp.tanh(cc), 0.0)
    h1 = jnp.maximum(
        jnp.dot(h, f1_ref[...], preferred_element_type=jnp.float32)
        + consts_ref[2:3, :],
        0.0,
    )
    h1_ref[...] = h1


_tc_gate = pl.pallas_call(
    _tc_gate_body,
    grid=(_GRID_N,),
    in_specs=[
        pl.BlockSpec((_RB, D), lambda i: (i, 0)),
        pl.BlockSpec((_RB, D), lambda i: (i, 0)),
        pl.BlockSpec((_RB, D), lambda i: (i, 0)),
        pl.BlockSpec((_RB, 1), lambda i: (i, 0)),
        pl.BlockSpec((D, D), lambda i: (0, 0)),
        pl.BlockSpec((D, D), lambda i: (0, 0)),
        pl.BlockSpec((D, D), lambda i: (0, 0)),
        pl.BlockSpec((3, D), lambda i: (0, 0)),
        pl.BlockSpec((DH, D), lambda i: (0, 0)),
    ],
    out_specs=pl.BlockSpec((_RB, D), lambda i: (i, 0)),
    out_shape=jax.ShapeDtypeStruct((N, D), jnp.float32),
)


_SB = 1024
_GRID_S = S // _SB


def _tc_head_body(pl_ref, pr_ref, w2a_ref, w2b_ref, w3_ref, wb_ref, wl_ref,
                  consts_ref, bin_ref, lead_ref):
    h2 = jnp.maximum(
        jnp.dot(pl_ref[:, 0:16], w2a_ref[...], preferred_element_type=jnp.float32)
        + jnp.dot(pr_ref[:, 0:16], w2b_ref[...], preferred_element_type=jnp.float32)
        + consts_ref[0:1, :],
        0.0,
    )
    h3 = jnp.maximum(
        jnp.dot(h2[:, 0:16], w3_ref[...], preferred_element_type=jnp.float32)
        + consts_ref[1:2, :],
        0.0,
    )
    h3s = h3[:, 0:8]
    bin_logit = (
        jnp.dot(h3s, wb_ref[...], preferred_element_type=jnp.float32)
        + consts_ref[2:3, :]
    )
    bin_ref[...] = _sigmoid(bin_logit[:, 0:1])
    logits = (
        jnp.dot(h3s, wl_ref[...], preferred_element_type=jnp.float32)
        + consts_ref[3:4, :]
    )
    m = jnp.max(logits, axis=1, keepdims=True)
    lse = jnp.log(jnp.sum(jnp.exp(logits - m), axis=1, keepdims=True)) + m
    lead_ref[...] = (logits - lse)[:, 0:LEAD + 1]


_tc_head = pl.pallas_call(
    _tc_head_body,
    grid=(_GRID_S,),
    in_specs=[
        pl.BlockSpec((_SB, D), lambda i: (i, 0)),
        pl.BlockSpec((_SB, D), lambda i: (i, 0)),
        pl.BlockSpec((16, D), lambda i: (0, 0)),
        pl.BlockSpec((16, D), lambda i: (0, 0)),
        pl.BlockSpec((16, D), lambda i: (0, 0)),
        pl.BlockSpec((8, D), lambda i: (0, 0)),
        pl.BlockSpec((8, D), lambda i: (0, 0)),
        pl.BlockSpec((4, D), lambda i: (0, 0)),
    ],
    out_specs=[
        pl.BlockSpec((_SB, 1), lambda i: (i, 0)),
        pl.BlockSpec((_SB, LEAD + 1), lambda i: (i, 0)),
    ],
    out_shape=[
        jax.ShapeDtypeStruct((S, 1), jnp.float32),
        jax.ShapeDtypeStruct((S, LEAD + 1), jnp.float32),
    ],
)


def _pad_cols(a, cols):
    return jnp.concatenate(
        [a, jnp.zeros((a.shape[0], cols - a.shape[1]), jnp.float32)], axis=1
    )


def kernel(x, params, edge_index, src, dst):
    p = params
    row = edge_index[0].astype(jnp.int32)
    col = edge_index[1].astype(jnp.int32)
    padn = N + (jnp.arange(EP - E, dtype=jnp.int32) % (NPAD - N))
    pad0 = jnp.zeros((EP - E,), jnp.int32)
    rowf = jnp.concatenate([row, pad0])
    colf = jnp.concatenate([col, padn])
    pidx = ((rowf << PSH) | colf).reshape(EP // CB, CB)
    rowd = jnp.concatenate([row, padn]).reshape(EP // CB, CB)
    zn = jnp.zeros((NPAD,), jnp.float32)
    z2 = jnp.zeros((NPAD, D), jnp.float32)
    sc_deg, sc_seg, sc_pair = _sc_kernels()

    degp = sc_deg(rowd, zn).reshape(NC, NPAD)[:, :N]
    xs, disc = _tc_scale(degp.T, x)
    s1p = sc_seg(xs, pidx, z2)
    t1, u1 = _tc_mid(s1p, disc)
    s2p = sc_seg(u1, pidx, z2)

    a0 = _pad_cols(jnp.concatenate(
        [p["W_xi"][0], p["W_xc"][0], p["W_xo"][0]], axis=1), D)
    a1 = _pad_cols(jnp.concatenate(
        [p["W_xi"][1], p["W_xc"][1], p["W_xo"][1]], axis=1), D)
    a2 = _pad_cols(jnp.concatenate(
        [p["W_xi"][2], p["W_xc"][2], p["W_xo"][2]], axis=1), D)
    bias_cat = jnp.concatenate([
        p["b_xi"] + p["b_hi"] + p["b_i"],
        p["b_xc"] + p["b_hc"] + p["b_c"],
        p["b_xo"] + p["b_ho"] + p["b_o"],
    ])
    wco_row = jnp.zeros((D,), jnp.float32).at[2 * DH:3 * DH].set(p["w_co"])
    f1b_row = jnp.zeros((D,), jnp.float32).at[0:16].set(p["fc1_b"])
    consts = jnp.stack([_pad_cols(bias_cat[None, :], D)[0], wco_row, f1b_row])
    f1 = _pad_cols(p["fc1_w"].T, D)

    h1 = _tc_gate(x, t1, s2p, disc, a0, a1, a2, consts, f1)

    sd = jnp.concatenate(
        [src.astype(jnp.int32), dst.astype(jnp.int32)]
    ).reshape(2 * (S // 128), 128)
    pair_l, pair_r = sc_pair(h1, sd)

    w2a = _pad_cols(p["fc2_w"][:, :16].T, D)
    w2b = _pad_cols(p["fc2_w"][:, 16:].T, D)
    w3 = _pad_cols(p["fc3_w"].T, D)
    wb = _pad_cols(p["bin_w"].T, D)
    wl = _pad_cols(p["lead_w"].T, D)
    b2r = jnp.zeros((D,), jnp.float32).at[0:16].set(p["fc2_b"])
    b3r = jnp.zeros((D,), jnp.float32).at[0:8].set(p["fc3_b"])
    bbr = jnp.zeros((D,), jnp.float32).at[0:1].set(p["bin_b"])
    blr = jnp.full((D,), -1e30, jnp.float32).at[0:LEAD + 1].set(p["lead_b"])
    hconsts = jnp.stack([b2r, b3r, bbr, blr])

    bin_h, lead_h = _tc_head(pair_l, pair_r, w2a, w2b, w3, wb, wl, hconsts)
    return (bin_h, lead_h)


# speedup vs baseline: 3.4549x; 3.4549x over previous
import functools

import jax
import jax.numpy as jnp
from jax import lax
from jax.experimental import pallas as pl
from jax.experimental.pallas import tpu as pltpu
from jax.experimental.pallas import tpu_sc as plsc

N = 10000
E = 320000
D = 128
DH = 32
S = 8192
LEAD = 48

NC = 2
NS = 16
NW = NC * NS

CB = 128
CPW = 80
EP = NW * CPW * CB
PSH = 14
NPAD = 10112
NPS = NPAD // NS


def _wid():
    return lax.axis_index("c") * NS + lax.axis_index("s")


def _sc_deg_body(row2_hbm, zn_hbm, out_hbm, idx_v, ones_v, acc):
    c = lax.axis_index("c")
    s = lax.axis_index("s")
    w = _wid()

    for i in range(CB // 16):
        ones_v[pl.ds(i * 16, 16)] = jnp.ones((16,), jnp.float32)

    @pl.when(s == 0)
    def _():
        pltpu.sync_copy(zn_hbm, acc)

    pltpu.sync_copy(row2_hbm.at[pl.ds(w * CPW, CPW), :], idx_v)
    plsc.subcore_barrier()

    @pl.loop(0, CPW)
    def _(j):
        pltpu.sync_copy(ones_v, acc.at[idx_v.at[j]], add=True)

    plsc.subcore_barrier()

    @pl.when(s == 0)
    def _():
        pltpu.sync_copy(acc, out_hbm.at[pl.ds(c * NPAD, NPAD)])


def _sc_seg_body(tbl_hbm, pidx_hbm, z2_hbm, out_hbm, pidx, rv0, rv1, cv,
                 buf0, buf1, acc, sg0, sg1):
    c = lax.axis_index("c")
    s = lax.axis_index("s")
    w = _wid()

    def unpack_rows(j, rv):
        for k in range(CB // 16):
            v = pidx[j, pl.ds(16 * k, 16)]
            rv[pl.ds(16 * k, 16)] = lax.shift_right_logical(v, PSH)

    def unpack_cols(j):
        for k in range(CB // 16):
            v = pidx[j, pl.ds(16 * k, 16)]
            cv[pl.ds(16 * k, 16)] = lax.bitwise_and(v, (1 << PSH) - 1)

    pltpu.sync_copy(z2_hbm.at[pl.ds(s * NPS, NPS), :], acc.at[pl.ds(s * NPS, NPS), :])
    pltpu.sync_copy(pidx_hbm.at[pl.ds(w * CPW, CPW), :], pidx)

    unpack_rows(0, rv0)
    pltpu.async_copy(tbl_hbm.at[rv0], buf0, sg0)
    unpack_rows(1, rv1)
    pltpu.async_copy(tbl_hbm.at[rv1], buf1, sg1)

    @pl.loop(0, CPW // 2)
    def _(i):
        j = 2 * i
        pltpu.make_async_copy(tbl_hbm.at[rv0], buf0, sg0).wait()
        unpack_cols(j)
        pltpu.sync_copy(buf0, acc.at[cv], add=True)

        @pl.when(j + 2 < CPW)
        def _():
            unpack_rows(j + 2, rv0)
            pltpu.async_copy(tbl_hbm.at[rv0], buf0, sg0)

        pltpu.make_async_copy(tbl_hbm.at[rv1], buf1, sg1).wait()
        unpack_cols(j + 1)
        pltpu.sync_copy(buf1, acc.at[cv], add=True)

        @pl.when(j + 3 < CPW)
        def _():
            unpack_rows(j + 3, rv1)
            pltpu.async_copy(tbl_hbm.at[rv1], buf1, sg1)

    plsc.subcore_barrier()
    pltpu.sync_copy(
        acc.at[pl.ds(s * NPS, NPS), :], out_hbm.at[c, pl.ds(s * NPS, NPS), :]
    )


_PPW = S // NW // 128


def _sc_pair_body(h1_hbm, sd_hbm, outL_hbm, outR_hbm, idx_v, buf, sem):
    w = _wid()
    pltpu.sync_copy(sd_hbm, idx_v)
    for t in range(2):
        out = outL_hbm if t == 0 else outR_hbm
        for j in range(_PPW):
            r = t * (S // 128) + w * _PPW + j
            pltpu.async_copy(h1_hbm.at[idx_v.at[r]], buf, sem).wait()
            pltpu.sync_copy(buf, out.at[pl.ds(w * _PPW * 128 + j * 128, 128), :])


@functools.cache
def _sc_kernels():
    mesh = plsc.VectorSubcoreMesh(
        core_axis_name="c", subcore_axis_name="s", num_cores=NC, num_subcores=NS
    )
    sc_deg = pl.kernel(
        _sc_deg_body,
        out_type=jax.ShapeDtypeStruct((NC * NPAD,), jnp.float32),
        mesh=mesh,
        scratch_types=[
            pltpu.VMEM((CPW, CB), jnp.int32),
            pltpu.VMEM((CB,), jnp.float32),
            pltpu.VMEM_SHARED((NPAD,), jnp.float32),
        ],
    )
    sc_seg = pl.kernel(
        _sc_seg_body,
        out_type=jax.ShapeDtypeStruct((NC, NPAD, D), jnp.float32),
        mesh=mesh,
        scratch_types=[
            pltpu.VMEM((CPW, CB), jnp.int32),
            pltpu.VMEM((CB,), jnp.int32),
            pltpu.VMEM((CB,), jnp.int32),
            pltpu.VMEM((CB,), jnp.int32),
            pltpu.VMEM((CB, D), jnp.float32),
            pltpu.VMEM((CB, D), jnp.float32),
            pltpu.VMEM_SHARED((NPAD, D), jnp.float32),
            pltpu.SemaphoreType.DMA,
            pltpu.SemaphoreType.DMA,
        ],
    )
    sc_pair = pl.kernel(
        _sc_pair_body,
        out_type=(
            jax.ShapeDtypeStruct((S, D), jnp.float32),
            jax.ShapeDtypeStruct((S, D), jnp.float32),
        ),
        mesh=mesh,
        scratch_types=[
            pltpu.VMEM((2 * (S // 128), 128), jnp.int32),
            pltpu.VMEM((128, D), jnp.float32),
            pltpu.SemaphoreType.DMA,
        ],
    )
    return sc_deg, sc_seg, sc_pair


_RB = 400
_GRID_N = N // _RB


def _sigmoid(v):
    return 1.0 / (1.0 + jnp.exp(-v))


def _tc_scale_body(degp_ref, x_ref, xs_ref, disc_ref):
    deg = jnp.sum(degp_ref[...], axis=1, keepdims=True)
    good = deg > 0.0
    dis = jnp.where(good, jax.lax.rsqrt(jnp.where(good, deg, 1.0)), 0.0)
    disc_ref[...] = dis
    xs_ref[...] = dis * x_ref[...]


_tc_scale = pl.pallas_call(
    _tc_scale_body,
    grid=(_GRID_N,),
    in_specs=[
        pl.BlockSpec((_RB, 2), lambda i: (i, 0)),
        pl.BlockSpec((_RB, D), lambda i: (i, 0)),
    ],
    out_specs=[
        pl.BlockSpec((_RB, D), lambda i: (i, 0)),
        pl.BlockSpec((_RB, 1), lambda i: (i, 0)),
    ],
    out_shape=[
        jax.ShapeDtypeStruct((N, D), jnp.float32),
        jax.ShapeDtypeStruct((N, 1), jnp.float32),
    ],
)


def _tc_mid_body(s1p_ref, disc_ref, t1_ref, u1_ref):
    P = s1p_ref[0] + s1p_ref[1]
    dis = disc_ref[...]
    t1 = -dis * P
    t1_ref[...] = t1
    u1_ref[...] = dis * t1


_tc_mid = pl.pallas_call(
    _tc_mid_body,
    grid=(_GRID_N,),
    in_specs=[
        pl.BlockSpec((NC, _RB, D), lambda i: (0, i, 0)),
        pl.BlockSpec((_RB, 1), lambda i: (i, 0)),
    ],
    out_specs=[
        pl.BlockSpec((_RB, D), lambda i: (i, 0)),
        pl.BlockSpec((_RB, D), lambda i: (i, 0)),
    ],
    out_shape=[
        jax.ShapeDtypeStruct((N, D), jnp.float32),
        jax.ShapeDtypeStruct((N, D), jnp.float32),
    ],
)


def _tc_gate_body(x_ref, t1_ref, s2p_ref, disc_ref, a0_ref, a1_ref, a2_ref,
                  consts_ref, f1_ref, h1_ref):
    x = x_ref[...]
    dis = disc_ref[...]
    t2 = -2.0 * dis * (s2p_ref[0] + s2p_ref[1]) - x
    pre = (
        jnp.dot(x, a0_ref[...], preferred_element_type=jnp.float32)
        + jnp.dot(t1_ref[...], a1_ref[...], preferred_element_type=jnp.float32)
        + jnp.dot(t2, a2_ref[...], preferred_element_type=jnp.float32)
        + consts_ref[0:1, :]
    )
    gi = _sigmoid(pre[:, 0:DH])
    gt = jnp.tanh(pre[:, DH:2 * DH])
    cc = gi * gt
    go = _sigmoid(pre[:, 2 * DH:3 * DH] + consts_ref[1:2, 2 * DH:3 * DH] * cc)
    h = jnp.maximum(go * jnp.tanh(cc), 0.0)
    h1 = jnp.maximum(
        jnp.dot(h, f1_ref[...], preferred_element_type=jnp.float32)
        + consts_ref[2:3, :],
        0.0,
    )
    h1_ref[...] = h1


_tc_gate = pl.pallas_call(
    _tc_gate_body,
    grid=(_GRID_N,),
    in_specs=[
        pl.BlockSpec((_RB, D), lambda i: (i, 0)),
        pl.BlockSpec((_RB, D), lambda i: (i, 0)),
        pl.BlockSpec((NC, _RB, D), lambda i: (0, i, 0)),
        pl.BlockSpec((_RB, 1), lambda i: (i, 0)),
        pl.BlockSpec((D, D), lambda i: (0, 0)),
        pl.BlockSpec((D, D), lambda i: (0, 0)),
        pl.BlockSpec((D, D), lambda i: (0, 0)),
        pl.BlockSpec((3, D), lambda i: (0, 0)),
        pl.BlockSpec((DH, D), lambda i: (0, 0)),
    ],
    out_specs=pl.BlockSpec((_RB, D), lambda i: (i, 0)),
    out_shape=jax.ShapeDtypeStruct((N, D), jnp.float32),
)


_SB = 1024
_GRID_S = S // _SB


def _tc_head_body(pl_ref, pr_ref, w2a_ref, w2b_ref, w3_ref, wb_ref, wl_ref,
                  consts_ref, bin_ref, lead_ref):
    h2 = jnp.maximum(
        jnp.dot(pl_ref[:, 0:16], w2a_ref[...], preferred_element_type=jnp.float32)
        + jnp.dot(pr_ref[:, 0:16], w2b_ref[...], preferred_element_type=jnp.float32)
        + consts_ref[0:1, :],
        0.0,
    )
    h3 = jnp.maximum(
        jnp.dot(h2[:, 0:16], w3_ref[...], preferred_element_type=jnp.float32)
        + consts_ref[1:2, :],
        0.0,
    )
    h3s = h3[:, 0:8]
    bin_logit = (
        jnp.dot(h3s, wb_ref[...], preferred_element_type=jnp.float32)
        + consts_ref[2:3, :]
    )
    bin_ref[...] = _sigmoid(bin_logit[:, 0:1])
    logits = (
        jnp.dot(h3s, wl_ref[...], preferred_element_type=jnp.float32)
        + consts_ref[3:4, :]
    )
    m = jnp.max(logits, axis=1, keepdims=True)
    lse = jnp.log(jnp.sum(jnp.exp(logits - m), axis=1, keepdims=True)) + m
    lead_ref[...] = (logits - lse)[:, 0:LEAD + 1]


_tc_head = pl.pallas_call(
    _tc_head_body,
    grid=(_GRID_S,),
    in_specs=[
        pl.BlockSpec((_SB, D), lambda i: (i, 0)),
        pl.BlockSpec((_SB, D), lambda i: (i, 0)),
        pl.BlockSpec((16, D), lambda i: (0, 0)),
        pl.BlockSpec((16, D), lambda i: (0, 0)),
        pl.BlockSpec((16, D), lambda i: (0, 0)),
        pl.BlockSpec((8, D), lambda i: (0, 0)),
        pl.BlockSpec((8, D), lambda i: (0, 0)),
        pl.BlockSpec((4, D), lambda i: (0, 0)),
    ],
    out_specs=[
        pl.BlockSpec((_SB, 1), lambda i: (i, 0)),
        pl.BlockSpec((_SB, LEAD + 1), lambda i: (i, 0)),
    ],
    out_shape=[
        jax.ShapeDtypeStruct((S, 1), jnp.float32),
        jax.ShapeDtypeStruct((S, LEAD + 1), jnp.float32),
    ],
)


def _pad_cols(a, cols):
    return jnp.concatenate(
        [a, jnp.zeros((a.shape[0], cols - a.shape[1]), jnp.float32)], axis=1
    )


def kernel(x, params, edge_index, src, dst):
    p = params
    row = edge_index[0].astype(jnp.int32)
    col = edge_index[1].astype(jnp.int32)
    padn = N + (jnp.arange(EP - E, dtype=jnp.int32) % (NPAD - N))
    pad0 = jnp.arange(EP - E, dtype=jnp.int32) % N
    rowf = jnp.concatenate([row, pad0])
    colf = jnp.concatenate([col, padn])
    pidx = ((rowf << PSH) | colf).reshape(EP // CB, CB)
    rowd = jnp.concatenate([row, padn]).reshape(EP // CB, CB)
    zn = jnp.zeros((NPAD,), jnp.float32)
    z2 = jnp.zeros((NPAD, D), jnp.float32)
    sc_deg, sc_seg, sc_pair = _sc_kernels()

    degp = sc_deg(rowd, zn).reshape(NC, NPAD)[:, :N]
    xs, disc = _tc_scale(degp.T, x)
    s1p = sc_seg(xs, pidx, z2)
    t1, u1 = _tc_mid(s1p, disc)
    s2p = sc_seg(u1, pidx, z2)

    a0 = _pad_cols(jnp.concatenate(
        [p["W_xi"][0], p["W_xc"][0], p["W_xo"][0]], axis=1), D)
    a1 = _pad_cols(jnp.concatenate(
        [p["W_xi"][1], p["W_xc"][1], p["W_xo"][1]], axis=1), D)
    a2 = _pad_cols(jnp.concatenate(
        [p["W_xi"][2], p["W_xc"][2], p["W_xo"][2]], axis=1), D)
    bias_cat = jnp.concatenate([
        p["b_xi"] + p["b_hi"] + p["b_i"],
        p["b_xc"] + p["b_hc"] + p["b_c"],
        p["b_xo"] + p["b_ho"] + p["b_o"],
    ])
    wco_row = jnp.zeros((D,), jnp.float32).at[2 * DH:3 * DH].set(p["w_co"])
    f1b_row = jnp.zeros((D,), jnp.float32).at[0:16].set(p["fc1_b"])
    consts = jnp.stack([_pad_cols(bias_cat[None, :], D)[0], wco_row, f1b_row])
    f1 = _pad_cols(p["fc1_w"].T, D)

    h1 = _tc_gate(x, t1, s2p, disc, a0, a1, a2, consts, f1)

    sd = jnp.concatenate(
        [src.astype(jnp.int32), dst.astype(jnp.int32)]
    ).reshape(2 * (S // 128), 128)
    pair_l, pair_r = sc_pair(h1, sd)

    w2a = _pad_cols(p["fc2_w"][:, :16].T, D)
    w2b = _pad_cols(p["fc2_w"][:, 16:].T, D)
    w3 = _pad_cols(p["fc3_w"].T, D)
    wb = _pad_cols(p["bin_w"].T, D)
    wl = _pad_cols(p["lead_w"].T, D)
    b2r = jnp.zeros((D,), jnp.float32).at[0:16].set(p["fc2_b"])
    b3r = jnp.zeros((D,), jnp.float32).at[0:8].set(p["fc3_b"])
    bbr = jnp.zeros((D,), jnp.float32).at[0:1].set(p["bin_b"])
    blr = jnp.full((D,), -1e30, jnp.float32).at[0:LEAD + 1].set(p["lead_b"])
    hconsts = jnp.stack([b2r, b3r, bbr, blr])

    bin_h, lead_h = _tc_head(pair_l, pair_r, w2a, w2b, w3, wb, wl, hconsts)
    return (bin_h, lead_h)


# speedup vs baseline: 3.7130x; 1.0747x over previous
import functools

import jax
import jax.numpy as jnp
from jax import lax
from jax.experimental import pallas as pl
from jax.experimental.pallas import tpu as pltpu
from jax.experimental.pallas import tpu_sc as plsc

N = 10000
E = 320000
D = 128
DH = 32
S = 8192
LEAD = 48

NC = 2
NS = 16
NW = NC * NS

CB = 128
CPW = 80
EP = NW * CPW * CB
PSH = 14
NPAD = 10112
NPS = NPAD // NS


def _wid():
    return lax.axis_index("c") * NS + lax.axis_index("s")


def _sc_deg_body(row2_hbm, zn_hbm, out_hbm, idx_v, ones_v, acc):
    c = lax.axis_index("c")
    s = lax.axis_index("s")
    w = _wid()

    for i in range(CB // 16):
        ones_v[pl.ds(i * 16, 16)] = jnp.ones((16,), jnp.float32)

    @pl.when(s == 0)
    def _():
        pltpu.sync_copy(zn_hbm, acc)

    pltpu.sync_copy(row2_hbm.at[pl.ds(w * CPW, CPW), :], idx_v)
    plsc.subcore_barrier()

    @pl.loop(0, CPW)
    def _(j):
        pltpu.sync_copy(ones_v, acc.at[idx_v.at[j]], add=True)

    plsc.subcore_barrier()

    @pl.when(s == 0)
    def _():
        pltpu.sync_copy(acc, out_hbm.at[pl.ds(c * NPAD, NPAD)])


def _sc_seg_body(tbl_hbm, pidx_hbm, z2_hbm, out_hbm, pidx, rv0, rv1, cv,
                 buf0, buf1, acc, sg0, sg1):
    c = lax.axis_index("c")
    s = lax.axis_index("s")
    w = _wid()

    def unpack_rows(j, rv):
        for k in range(CB // 16):
            v = pidx[j, pl.ds(16 * k, 16)]
            rv[pl.ds(16 * k, 16)] = lax.shift_right_logical(v, PSH)

    def unpack_cols(j):
        for k in range(CB // 16):
            v = pidx[j, pl.ds(16 * k, 16)]
            cv[pl.ds(16 * k, 16)] = lax.bitwise_and(v, (1 << PSH) - 1)

    pltpu.sync_copy(z2_hbm.at[pl.ds(s * NPS, NPS), :], acc.at[pl.ds(s * NPS, NPS), :])
    pltpu.sync_copy(pidx_hbm.at[pl.ds(w * CPW, CPW), :], pidx)

    unpack_rows(0, rv0)
    pltpu.async_copy(tbl_hbm.at[rv0], buf0, sg0)
    unpack_rows(1, rv1)
    pltpu.async_copy(tbl_hbm.at[rv1], buf1, sg1)

    @pl.loop(0, CPW // 2)
    def _(i):
        j = 2 * i
        pltpu.make_async_copy(tbl_hbm.at[rv0], buf0, sg0).wait()
        unpack_cols(j)
        pltpu.sync_copy(buf0, acc.at[cv], add=True)

        @pl.when(j + 2 < CPW)
        def _():
            unpack_rows(j + 2, rv0)
            pltpu.async_copy(tbl_hbm.at[rv0], buf0, sg0)

        pltpu.make_async_copy(tbl_hbm.at[rv1], buf1, sg1).wait()
        unpack_cols(j + 1)
        pltpu.sync_copy(buf1, acc.at[cv], add=True)

        @pl.when(j + 3 < CPW)
        def _():
            unpack_rows(j + 3, rv1)
            pltpu.async_copy(tbl_hbm.at[rv1], buf1, sg1)

    plsc.subcore_barrier()
    pltpu.sync_copy(
        acc.at[pl.ds(s * NPS, NPS), :], out_hbm.at[c, pl.ds(s * NPS, NPS), :]
    )


_PPW = S // NW // 128


def _sc_pair_body(h1_hbm, sd_hbm, outL_hbm, outR_hbm, idx_v, buf, sem):
    w = _wid()
    pltpu.sync_copy(sd_hbm, idx_v)
    for t in range(2):
        out = outL_hbm if t == 0 else outR_hbm
        for j in range(_PPW):
            r = t * (S // 128) + w * _PPW + j
            pltpu.async_copy(h1_hbm.at[idx_v.at[r]], buf, sem).wait()
            pltpu.sync_copy(buf, out.at[pl.ds(w * _PPW * 128 + j * 128, 128), :])


@functools.cache
def _sc_kernels():
    mesh = plsc.VectorSubcoreMesh(
        core_axis_name="c", subcore_axis_name="s", num_cores=NC, num_subcores=NS
    )
    sc_deg = pl.kernel(
        _sc_deg_body,
        out_type=jax.ShapeDtypeStruct((NC * NPAD,), jnp.float32),
        mesh=mesh,
        scratch_types=[
            pltpu.VMEM((CPW, CB), jnp.int32),
            pltpu.VMEM((CB,), jnp.float32),
            pltpu.VMEM_SHARED((NPAD,), jnp.float32),
        ],
    )
    sc_seg = pl.kernel(
        _sc_seg_body,
        out_type=jax.ShapeDtypeStruct((NC, NPAD, D), jnp.float32),
        mesh=mesh,
        scratch_types=[
            pltpu.VMEM((CPW, CB), jnp.int32),
            pltpu.VMEM((CB,), jnp.int32),
            pltpu.VMEM((CB,), jnp.int32),
            pltpu.VMEM((CB,), jnp.int32),
            pltpu.VMEM((CB, D), jnp.float32),
            pltpu.VMEM((CB, D), jnp.float32),
            pltpu.VMEM_SHARED((NPAD, D), jnp.float32),
            pltpu.SemaphoreType.DMA,
            pltpu.SemaphoreType.DMA,
        ],
    )
    sc_pair = pl.kernel(
        _sc_pair_body,
        out_type=(
            jax.ShapeDtypeStruct((S, D), jnp.float32),
            jax.ShapeDtypeStruct((S, D), jnp.float32),
        ),
        mesh=mesh,
        scratch_types=[
            pltpu.VMEM((2 * (S // 128), 128), jnp.int32),
            pltpu.VMEM((128, D), jnp.float32),
            pltpu.SemaphoreType.DMA,
        ],
    )
    return sc_deg, sc_seg, sc_pair


_RB = 1000
_GRID_N = N // _RB


def _sigmoid(v):
    return 1.0 / (1.0 + jnp.exp(-v))


def _tc_scale_body(disc_ref, x_ref, xs_ref):
    xs_ref[...] = disc_ref[...] * x_ref[...]


_tc_scale = pl.pallas_call(
    _tc_scale_body,
    grid=(_GRID_N,),
    in_specs=[
        pl.BlockSpec((_RB, 1), lambda i: (i, 0)),
        pl.BlockSpec((_RB, D), lambda i: (i, 0)),
    ],
    out_specs=pl.BlockSpec((_RB, D), lambda i: (i, 0)),
    out_shape=jax.ShapeDtypeStruct((N, D), jnp.float32),
)


def _tc_mid_body(s1p_ref, disc_ref, t1_ref, u1_ref):
    P = s1p_ref[0] + s1p_ref[1]
    dis = disc_ref[...]
    t1 = -dis * P
    t1_ref[...] = t1
    u1_ref[...] = dis * t1


_tc_mid = pl.pallas_call(
    _tc_mid_body,
    grid=(_GRID_N,),
    in_specs=[
        pl.BlockSpec((NC, _RB, D), lambda i: (0, i, 0)),
        pl.BlockSpec((_RB, 1), lambda i: (i, 0)),
    ],
    out_specs=[
        pl.BlockSpec((_RB, D), lambda i: (i, 0)),
        pl.BlockSpec((_RB, D), lambda i: (i, 0)),
    ],
    out_shape=[
        jax.ShapeDtypeStruct((N, D), jnp.float32),
        jax.ShapeDtypeStruct((N, D), jnp.float32),
    ],
)


def _tc_gate_body(x_ref, t1_ref, s2p_ref, disc_ref, a0_ref, a1_ref, a2_ref,
                  consts_ref, f1_ref, h1_ref):
    x = x_ref[...]
    dis = disc_ref[...]
    t2 = -2.0 * dis * (s2p_ref[0] + s2p_ref[1]) - x
    pre = (
        jnp.dot(x, a0_ref[...], preferred_element_type=jnp.float32)
        + jnp.dot(t1_ref[...], a1_ref[...], preferred_element_type=jnp.float32)
        + jnp.dot(t2, a2_ref[...], preferred_element_type=jnp.float32)
        + consts_ref[0:1, :]
    )
    gi = _sigmoid(pre[:, 0:DH])
    gt = jnp.tanh(pre[:, DH:2 * DH])
    cc = gi * gt
    go = _sigmoid(pre[:, 2 * DH:3 * DH] + consts_ref[1:2, 2 * DH:3 * DH] * cc)
    h = jnp.maximum(go * jnp.tanh(cc), 0.0)
    h1 = jnp.maximum(
        jnp.dot(h, f1_ref[...], preferred_element_type=jnp.float32)
        + consts_ref[2:3, :],
        0.0,
    )
    h1_ref[...] = h1


_tc_gate = pl.pallas_call(
    _tc_gate_body,
    grid=(_GRID_N,),
    in_specs=[
        pl.BlockSpec((_RB, D), lambda i: (i, 0)),
        pl.BlockSpec((_RB, D), lambda i: (i, 0)),
        pl.BlockSpec((NC, _RB, D), lambda i: (0, i, 0)),
        pl.BlockSpec((_RB, 1), lambda i: (i, 0)),
        pl.BlockSpec((D, D), lambda i: (0, 0)),
        pl.BlockSpec((D, D), lambda i: (0, 0)),
        pl.BlockSpec((D, D), lambda i: (0, 0)),
        pl.BlockSpec((3, D), lambda i: (0, 0)),
        pl.BlockSpec((DH, D), lambda i: (0, 0)),
    ],
    out_specs=pl.BlockSpec((_RB, D), lambda i: (i, 0)),
    out_shape=jax.ShapeDtypeStruct((N, D), jnp.float32),
)


_SB = 2048
_GRID_S = S // _SB


def _tc_head_body(pl_ref, pr_ref, w2a_ref, w2b_ref, w3_ref, wb_ref, wl_ref,
                  consts_ref, bin_ref, lead_ref):
    h2 = jnp.maximum(
        jnp.dot(pl_ref[:, 0:16], w2a_ref[...], preferred_element_type=jnp.float32)
        + jnp.dot(pr_ref[:, 0:16], w2b_ref[...], preferred_element_type=jnp.float32)
        + consts_ref[0:1, :],
        0.0,
    )
    h3 = jnp.maximum(
        jnp.dot(h2[:, 0:16], w3_ref[...], preferred_element_type=jnp.float32)
        + consts_ref[1:2, :],
        0.0,
    )
    h3s = h3[:, 0:8]
    bin_logit = (
        jnp.dot(h3s, wb_ref[...], preferred_element_type=jnp.float32)
        + consts_ref[2:3, :]
    )
    bin_ref[...] = _sigmoid(bin_logit[:, 0:1])
    logits = (
        jnp.dot(h3s, wl_ref[...], preferred_element_type=jnp.float32)
        + consts_ref[3:4, :]
    )
    m = jnp.max(logits, axis=1, keepdims=True)
    lse = jnp.log(jnp.sum(jnp.exp(logits - m), axis=1, keepdims=True)) + m
    lead_ref[...] = (logits - lse)[:, 0:LEAD + 1]


_tc_head = pl.pallas_call(
    _tc_head_body,
    grid=(_GRID_S,),
    in_specs=[
        pl.BlockSpec((_SB, D), lambda i: (i, 0)),
        pl.BlockSpec((_SB, D), lambda i: (i, 0)),
        pl.BlockSpec((16, D), lambda i: (0, 0)),
        pl.BlockSpec((16, D), lambda i: (0, 0)),
        pl.BlockSpec((16, D), lambda i: (0, 0)),
        pl.BlockSpec((8, D), lambda i: (0, 0)),
        pl.BlockSpec((8, D), lambda i: (0, 0)),
        pl.BlockSpec((4, D), lambda i: (0, 0)),
    ],
    out_specs=[
        pl.BlockSpec((_SB, 1), lambda i: (i, 0)),
        pl.BlockSpec((_SB, LEAD + 1), lambda i: (i, 0)),
    ],
    out_shape=[
        jax.ShapeDtypeStruct((S, 1), jnp.float32),
        jax.ShapeDtypeStruct((S, LEAD + 1), jnp.float32),
    ],
)


def _pad_cols(a, cols):
    return jnp.concatenate(
        [a, jnp.zeros((a.shape[0], cols - a.shape[1]), jnp.float32)], axis=1
    )


def kernel(x, params, edge_index, src, dst):
    p = params
    row = edge_index[0].astype(jnp.int32)
    col = edge_index[1].astype(jnp.int32)
    padn = N + (jnp.arange(EP - E, dtype=jnp.int32) % (NPAD - N))
    pad0 = jnp.arange(EP - E, dtype=jnp.int32) % N
    rowf = jnp.concatenate([row, pad0])
    colf = jnp.concatenate([col, padn])
    pidx = ((rowf << PSH) | colf).reshape(EP // CB, CB)
    rowd = jnp.concatenate([row, padn]).reshape(EP // CB, CB)
    zn = jnp.zeros((NPAD,), jnp.float32)
    z2 = jnp.zeros((NPAD, D), jnp.float32)
    sc_deg, sc_seg, sc_pair = _sc_kernels()

    degp = sc_deg(rowd, zn).reshape(NC, NPAD)[:, :N]
    deg = degp[0] + degp[1]
    disc = jnp.where(
        deg > 0.0, jax.lax.rsqrt(jnp.where(deg > 0.0, deg, 1.0)), 0.0
    )[:, None]
    xs = _tc_scale(disc, x)
    s1p = sc_seg(xs, pidx, z2)
    t1, u1 = _tc_mid(s1p, disc)
    s2p = sc_seg(u1, pidx, z2)

    a0 = _pad_cols(jnp.concatenate(
        [p["W_xi"][0], p["W_xc"][0], p["W_xo"][0]], axis=1), D)
    a1 = _pad_cols(jnp.concatenate(
        [p["W_xi"][1], p["W_xc"][1], p["W_xo"][1]], axis=1), D)
    a2 = _pad_cols(jnp.concatenate(
        [p["W_xi"][2], p["W_xc"][2], p["W_xo"][2]], axis=1), D)
    bias_cat = jnp.concatenate([
        p["b_xi"] + p["b_hi"] + p["b_i"],
        p["b_xc"] + p["b_hc"] + p["b_c"],
        p["b_xo"] + p["b_ho"] + p["b_o"],
    ])
    wco_row = jnp.zeros((D,), jnp.float32).at[2 * DH:3 * DH].set(p["w_co"])
    f1b_row = jnp.zeros((D,), jnp.float32).at[0:16].set(p["fc1_b"])
    consts = jnp.stack([_pad_cols(bias_cat[None, :], D)[0], wco_row, f1b_row])
    f1 = _pad_cols(p["fc1_w"].T, D)

    h1 = _tc_gate(x, t1, s2p, disc, a0, a1, a2, consts, f1)

    sd = jnp.concatenate(
        [src.astype(jnp.int32), dst.astype(jnp.int32)]
    ).reshape(2 * (S // 128), 128)
    pair_l, pair_r = sc_pair(h1, sd)

    w2a = _pad_cols(p["fc2_w"][:, :16].T, D)
    w2b = _pad_cols(p["fc2_w"][:, 16:].T, D)
    w3 = _pad_cols(p["fc3_w"].T, D)
    wb = _pad_cols(p["bin_w"].T, D)
    wl = _pad_cols(p["lead_w"].T, D)
    b2r = jnp.zeros((D,), jnp.float32).at[0:16].set(p["fc2_b"])
    b3r = jnp.zeros((D,), jnp.float32).at[0:8].set(p["fc3_b"])
    bbr = jnp.zeros((D,), jnp.float32).at[0:1].set(p["bin_b"])
    blr = jnp.full((D,), -1e30, jnp.float32).at[0:LEAD + 1].set(p["lead_b"])
    hconsts = jnp.stack([b2r, b3r, bbr, blr])

    bin_h, lead_h = _tc_head(pair_l, pair_r, w2a, w2b, w3, wb, wl, hconsts)
    return (bin_h, lead_h)
